# Initial kernel scaffold; baseline (speedup 1.0000x reference)
import jax
import jax.numpy as jnp
from jax.experimental import pallas as pl


def kernel(x, Wg, bg, W1, b1, W2, b2):
    raise NotImplementedError("write your pallas kernel here")


# speedup vs baseline: 1.3594x; 1.3594x over previous
import functools

import jax
import jax.numpy as jnp
from jax import lax
from jax.experimental import pallas as pl
from jax.experimental.pallas import tpu as pltpu

NUM_EXPERT = 16
TOP_K = 2
D_MODEL = 1024
D_FF = 1024
N_TOKENS = 8192

E_PAD = 128
NEG_BIG = -1e30


def _gating_body(x_ref, wg_ref, bg_ref, sm_ref):
    x = x_ref[...]
    logits = jnp.dot(x, wg_ref[...], preferred_element_type=jnp.float32)
    logits = logits + bg_ref[...]
    bt = logits.shape[0]
    lane = lax.broadcasted_iota(jnp.int32, (bt, E_PAD), 1)
    v1 = jnp.max(logits, axis=1, keepdims=True)
    c1 = jnp.where(logits >= v1, lane, E_PAD)
    i1 = jnp.min(c1, axis=1, keepdims=True)
    oh1 = (lane == i1)
    logits2 = jnp.where(oh1, NEG_BIG, logits)
    v2 = jnp.max(logits2, axis=1, keepdims=True)
    c2 = jnp.where(logits2 >= v2, lane, E_PAD)
    i2 = jnp.min(c2, axis=1, keepdims=True)
    oh2 = (lane == i2)
    e2 = jnp.exp(v2 - v1)
    s1 = 1.0 / (1.0 + e2)
    s2 = 1.0 - s1
    sm_ref[...] = jnp.where(oh1, s1, 0.0) + jnp.where(oh2, s2, 0.0)


def _gating(x, wg_pad, bg_pad, block_t=2048):
    t = x.shape[0]
    return pl.pallas_call(
        _gating_body,
        grid=(t // block_t,),
        in_specs=[
            pl.BlockSpec((block_t, D_MODEL), lambda i: (i, 0)),
            pl.BlockSpec((D_MODEL, E_PAD), lambda i: (0, 0)),
            pl.BlockSpec((1, E_PAD), lambda i: (0, 0)),
        ],
        out_specs=pl.BlockSpec((block_t, E_PAD), lambda i: (i, 0)),
        out_shape=jax.ShapeDtypeStruct((t, E_PAD), jnp.float32),
    )(x, wg_pad, bg_pad)


def _moe_body(xb_ref, w1_ref, b1_ref, w2_ref, b2_ref, sm_ref, out_ref):
    e = pl.program_id(1)
    x = xb_ref[...]
    h = jnp.dot(x, w1_ref[0], preferred_element_type=jnp.float32)
    h = jnp.maximum(h + b1_ref[0], 0.0).astype(jnp.bfloat16)
    y = jnp.dot(h, w2_ref[0], preferred_element_type=jnp.float32)
    y = y + b2_ref[0]
    bt = y.shape[0]
    lane = lax.broadcasted_iota(jnp.int32, (bt, E_PAD), 1)
    m = jnp.sum(jnp.where(lane == e, sm_ref[...], 0.0), axis=1, keepdims=True)

    @pl.when(e == 0)
    def _():
        out_ref[...] = jnp.zeros_like(out_ref)

    out_ref[...] += m * y


def _moe(xb, w1b, b1, w2b, b2, sm, block_t=2048):
    t = xb.shape[0]
    return pl.pallas_call(
        _moe_body,
        grid=(t // block_t, NUM_EXPERT),
        in_specs=[
            pl.BlockSpec((block_t, D_MODEL), lambda i, e: (i, 0)),
            pl.BlockSpec((1, D_MODEL, D_FF), lambda i, e: (e, 0, 0)),
            pl.BlockSpec((1, 1, D_FF), lambda i, e: (e, 0, 0)),
            pl.BlockSpec((1, D_FF, D_MODEL), lambda i, e: (e, 0, 0)),
            pl.BlockSpec((1, 1, D_MODEL), lambda i, e: (e, 0, 0)),
            pl.BlockSpec((block_t, E_PAD), lambda i, e: (i, 0)),
        ],
        out_specs=pl.BlockSpec((block_t, D_MODEL), lambda i, e: (i, 0)),
        out_shape=jax.ShapeDtypeStruct((t, D_MODEL), jnp.float32),
        compiler_params=pltpu.CompilerParams(
            dimension_semantics=("parallel", "arbitrary"),
        ),
    )(xb, w1b, b1, w2b, b2, sm)


def kernel(x, Wg, bg, W1, b1, W2, b2):
    wg_pad = jnp.zeros((D_MODEL, E_PAD), jnp.float32).at[:, :NUM_EXPERT].set(Wg)
    bg_pad = jnp.full((1, E_PAD), NEG_BIG, jnp.float32).at[0, :NUM_EXPERT].set(bg)
    sm = _gating(x, wg_pad, bg_pad)
    xb = x.astype(jnp.bfloat16)
    w1b = W1.astype(jnp.bfloat16)
    w2b = W2.astype(jnp.bfloat16)
    return _moe(xb, w1b, b1[:, None, :], w2b, b2[:, None, :], sm)
